# baseline (device time: 225980 ns/iter reference)
import jax
import jax.numpy as jnp
from jax import lax
from jax.experimental import pallas as pl
from jax.experimental.pallas import tpu as pltpu

N_DEV = 4
B, S, C = 4, 2048, 1024
OC = 1024
HC = OC // 2
KT = 4
CHUNK = S // N_DEV
HOPS = N_DEV - 1


def _compute_body(x_ref, k_ref, Wp_ref, out_ref):
    x = x_ref[0]
    xp = jnp.concatenate(
        [jnp.zeros((KT - 1, C), jnp.float32), x], axis=0)
    acc = xp[KT - 1:, :] * k_ref[KT - 1, :]
    for t in range(KT - 1):
        acc = acc + xp[t:t + S, :] * k_ref[t, :]
    a = acc * (1.0 / (1.0 + jnp.exp(-acc)))
    out_ref[0] = jnp.dot(
        a, Wp_ref[...], preferred_element_type=jnp.float32
    ).astype(jnp.bfloat16)


def _local_compute(x, k, Wp):
    return pl.pallas_call(
        _compute_body,
        grid=(B,),
        in_specs=[
            pl.BlockSpec((1, S, C), lambda b: (b, 0, 0)),
            pl.BlockSpec((KT, C), lambda b: (0, 0)),
            pl.BlockSpec((C, OC), lambda b: (0, 0)),
        ],
        out_specs=pl.BlockSpec((1, S, OC), lambda b: (b, 0, 0)),
        out_shape=jax.ShapeDtypeStruct((B, S, OC), jnp.bfloat16),
        compiler_params=pltpu.CompilerParams(
            vmem_limit_bytes=60 * 1024 * 1024,
        ),
    )(x, k, Wp)


def _ar_body(part_ref, out_ref,
             accA, accB, rsA_recv, rsB_recv, agA_recv, agB_recv,
             stageA, stageB,
             loadA_sem, loadB_sem, storeA_sem, storeB_sem,
             rsA_send_s, rsA_recv_s, rsB_send_s, rsB_recv_s,
             agA_send_s, agA_recv_s, agB_send_s, agB_recv_s):
    p = lax.axis_index("i")
    right = jnp.mod(p + 1, N_DEV)
    left = jnp.mod(p + N_DEV - 1, N_DEV)

    def pchunk(c, off):
        return part_ref.at[:, pl.ds(c * CHUNK, CHUNK), pl.ds(off, HC)]

    def ochunk(c, off):
        return out_ref.at[:, pl.ds(c * CHUNK, CHUNK), pl.ds(off, HC)]

    def rdma(src, dst, send_sem, recv_sem, dev):
        return pltpu.make_async_remote_copy(
            src_ref=src, dst_ref=dst, send_sem=send_sem, recv_sem=recv_sem,
            device_id=(dev,), device_id_type=pl.DeviceIdType.MESH,
        )

    def load(c, off, acc, slot, sem):
        cp = pltpu.make_async_copy(pchunk(c, off), acc.at[slot], sem.at[slot])
        cp.start()
        return cp

    ldA0 = load(p, 0, accA, 0, loadA_sem)
    ldB0 = load(p, HC, accB, 0, loadB_sem)
    ldA = load(jnp.mod(p - 1, N_DEV), 0, accA, 1, loadA_sem)
    ldB = load(jnp.mod(p + 1, N_DEV), HC, accB, 1, loadB_sem)

    barrier = pltpu.get_barrier_semaphore()
    for nbr in (left, right):
        pl.semaphore_signal(barrier, inc=1, device_id=(nbr,),
                            device_id_type=pl.DeviceIdType.MESH)
    pl.semaphore_wait(barrier, 2)

    rdA = [rdma(accA.at[h % 2], rsA_recv.at[h],
                rsA_send_s.at[h], rsA_recv_s.at[h], right)
           for h in range(HOPS)]
    rdB = [rdma(accB.at[h % 2], rsB_recv.at[h],
                rsB_send_s.at[h], rsB_recv_s.at[h], left)
           for h in range(HOPS)]

    ldA0.wait()
    ldB0.wait()
    rdA[0].start()
    rdB[0].start()

    for h in range(HOPS):
        nxt = (h + 1) % 2
        ldA.wait()
        ldB.wait()
        rdA[h].wait_recv()
        accA[nxt] = accA[nxt] + rsA_recv[h]
        if h < HOPS - 1:
            rdA[h + 1].start()
        rdB[h].wait_recv()
        accB[nxt] = accB[nxt] + rsB_recv[h]
        if h < HOPS - 1:
            rdB[h + 1].start()
            rdA[h].wait_send()
            rdB[h].wait_send()
            ldA = load(jnp.mod(p - h - 2, N_DEV), 0, accA, h % 2, loadA_sem)
            ldB = load(jnp.mod(p + h + 2, N_DEV), HC, accB, h % 2, loadB_sem)

    fin = HOPS % 2
    agA = [rdma(accA.at[fin] if g == 0 else agA_recv.at[g - 1],
                agA_recv.at[g], agA_send_s.at[g], agA_recv_s.at[g], right)
           for g in range(HOPS)]
    agB = [rdma(accB.at[fin] if g == 0 else agB_recv.at[g - 1],
                agB_recv.at[g], agB_send_s.at[g], agB_recv_s.at[g], left)
           for g in range(HOPS)]

    agA[0].start()
    agB[0].start()
    rdA[HOPS - 1].wait_send()
    rdB[HOPS - 1].wait_send()

    stageA[...] = accA[fin].astype(jnp.float32)
    stA = pltpu.make_async_copy(
        stageA, ochunk(jnp.mod(p + 1, N_DEV), 0), storeA_sem)
    stA.start()
    stageB[...] = accB[fin].astype(jnp.float32)
    stB = pltpu.make_async_copy(
        stageB, ochunk(jnp.mod(p - 1, N_DEV), HC), storeB_sem)
    stB.start()

    for g in range(HOPS):
        agA[g].wait_recv()
        if g < HOPS - 1:
            agA[g + 1].start()
        stA.wait()
        stageA[...] = agA_recv[g].astype(jnp.float32)
        stA = pltpu.make_async_copy(
            stageA, ochunk(jnp.mod(p - g, N_DEV), 0), storeA_sem)
        stA.start()
        agB[g].wait_recv()
        if g < HOPS - 1:
            agB[g + 1].start()
        stB.wait()
        stageB[...] = agB_recv[g].astype(jnp.float32)
        stB = pltpu.make_async_copy(
            stageB, ochunk(jnp.mod(p + g, N_DEV), HC), storeB_sem)
        stB.start()

    for g in range(HOPS):
        agA[g].wait_send()
        agB[g].wait_send()
    stA.wait()
    stB.wait()


def _all_reduce(part):
    return pl.pallas_call(
        _ar_body,
        in_specs=[pl.BlockSpec(memory_space=pl.ANY)],
        out_specs=pl.BlockSpec(memory_space=pl.ANY),
        out_shape=jax.ShapeDtypeStruct((B, S, OC), jnp.float32),
        scratch_shapes=[
            pltpu.VMEM((2, B, CHUNK, HC), jnp.bfloat16),
            pltpu.VMEM((2, B, CHUNK, HC), jnp.bfloat16),
            pltpu.VMEM((HOPS, B, CHUNK, HC), jnp.bfloat16),
            pltpu.VMEM((HOPS, B, CHUNK, HC), jnp.bfloat16),
            pltpu.VMEM((HOPS, B, CHUNK, HC), jnp.bfloat16),
            pltpu.VMEM((HOPS, B, CHUNK, HC), jnp.bfloat16),
            pltpu.VMEM((B, CHUNK, HC), jnp.float32),
            pltpu.VMEM((B, CHUNK, HC), jnp.float32),
            pltpu.SemaphoreType.DMA((2,)),
            pltpu.SemaphoreType.DMA((2,)),
            pltpu.SemaphoreType.DMA,
            pltpu.SemaphoreType.DMA,
            pltpu.SemaphoreType.DMA((HOPS,)),
            pltpu.SemaphoreType.DMA((HOPS,)),
            pltpu.SemaphoreType.DMA((HOPS,)),
            pltpu.SemaphoreType.DMA((HOPS,)),
            pltpu.SemaphoreType.DMA((HOPS,)),
            pltpu.SemaphoreType.DMA((HOPS,)),
            pltpu.SemaphoreType.DMA((HOPS,)),
            pltpu.SemaphoreType.DMA((HOPS,)),
        ],
        compiler_params=pltpu.CompilerParams(
            collective_id=0,
            vmem_limit_bytes=60 * 1024 * 1024,
        ),
    )(part)


def kernel(x, k, Wp):
    part = _local_compute(x, k, Wp)
    return _all_reduce(part)


# device time: 188293 ns/iter; 1.2002x vs baseline; 1.2002x over previous
import jax
import jax.numpy as jnp
from jax import lax
from jax.experimental import pallas as pl
from jax.experimental.pallas import tpu as pltpu

N_DEV = 4
B, S, C = 4, 2048, 1024
OC = 1024
HC = OC // 2
KT = 4
CHUNK = S // N_DEV
HALO = 8
OFF = HALO - KT + 1
HOPS = N_DEV - 1


def _body(x_ref, k_ref, Wp_ref, out_ref,
          partA, partB, rsA_recv, rsB_recv, agA_recv, agB_recv,
          stageA, stageB, xbuf,
          x_sem, storeA_sem, storeB_sem,
          rsA_send_s, rsA_recv_s, rsB_send_s, rsB_recv_s,
          agA_send_s, agA_recv_s, agB_send_s, agB_recv_s):
    p = lax.axis_index("i")
    right = jnp.mod(p + 1, N_DEV)
    left = jnp.mod(p + N_DEV - 1, N_DEV)

    barrier = pltpu.get_barrier_semaphore()
    for nbr in (left, right):
        pl.semaphore_signal(barrier, inc=1, device_id=(nbr,),
                            device_id_type=pl.DeviceIdType.MESH)
    pl.semaphore_wait(barrier, 2)

    def start_load(cglob, b, slot):
        halo_start = pl.multiple_of(
            jnp.maximum(cglob * CHUNK - HALO, 0), HALO)
        halo = pltpu.make_async_copy(
            x_ref.at[b, pl.ds(halo_start, HALO), :],
            xbuf.at[slot, pl.ds(0, HALO), :], x_sem.at[slot])
        main = pltpu.make_async_copy(
            x_ref.at[b, pl.ds(cglob * CHUNK, CHUNK), :],
            xbuf.at[slot, pl.ds(HALO, CHUNK), :], x_sem.at[slot])
        halo.start()
        main.start()
        return (halo, main, slot)

    def wait_load(ld, cglob):
        halo, main, slot = ld
        halo.wait()
        main.wait()

        @pl.when(cglob == 0)
        def _():
            xbuf[slot, :HALO, :] = jnp.zeros((HALO, C), jnp.float32)

    def compute_chunk(j):
        cglob = jnp.mod(p + j, N_DEV)
        cur = start_load(cglob, 0, 0)
        for b in range(B):
            nxt = start_load(cglob, b + 1, (b + 1) % 2) if b < B - 1 else None
            wait_load(cur, cglob)
            xv = xbuf[b % 2]
            acc = xv[OFF + KT - 1: OFF + KT - 1 + CHUNK, :] * k_ref[KT - 1, :]
            for t in range(KT - 1):
                acc = acc + xv[OFF + t: OFF + t + CHUNK, :] * k_ref[t, :]
            a = acc * (1.0 / (1.0 + jnp.exp(-acc)))
            r = jnp.dot(a, Wp_ref[...], preferred_element_type=jnp.float32)
            partA[j, b, :, :] = r[:, :HC].astype(jnp.bfloat16)
            partB[j, b, :, :] = r[:, HC:].astype(jnp.bfloat16)
            cur = nxt

    def rdma(src, dst, send_sem, recv_sem, dev):
        return pltpu.make_async_remote_copy(
            src_ref=src, dst_ref=dst, send_sem=send_sem, recv_sem=recv_sem,
            device_id=(dev,), device_id_type=pl.DeviceIdType.MESH,
        )

    def ochunk(c, off):
        return out_ref.at[:, pl.ds(c * CHUNK, CHUNK), pl.ds(off, HC)]

    rdA = [rdma(partA.at[(N_DEV - h) % N_DEV], rsA_recv.at[h],
                rsA_send_s.at[h], rsA_recv_s.at[h], right)
           for h in range(HOPS)]
    rdB = [rdma(partB.at[h], rsB_recv.at[h],
                rsB_send_s.at[h], rsB_recv_s.at[h], left)
           for h in range(HOPS)]

    compute_chunk(0)
    rdA[0].start()
    rdB[0].start()
    compute_chunk(N_DEV - 1)
    compute_chunk(1)

    for h in range(HOPS):
        sA = N_DEV - 1 - h
        sB = h + 1
        rdA[h].wait_recv()
        partA[sA] = partA[sA] + rsA_recv[h]
        if h < HOPS - 1:
            rdA[h + 1].start()
        rdB[h].wait_recv()
        partB[sB] = partB[sB] + rsB_recv[h]
        if h < HOPS - 1:
            rdB[h + 1].start()
            rdA[h].wait_send()
            rdB[h].wait_send()
        if h == 0:
            compute_chunk(2)

    agA = [rdma(partA.at[1] if g == 0 else agA_recv.at[g - 1],
                agA_recv.at[g], agA_send_s.at[g], agA_recv_s.at[g], right)
           for g in range(HOPS)]
    agB = [rdma(partB.at[N_DEV - 1] if g == 0 else agB_recv.at[g - 1],
                agB_recv.at[g], agB_send_s.at[g], agB_recv_s.at[g], left)
           for g in range(HOPS)]

    agA[0].start()
    agB[0].start()
    rdA[HOPS - 1].wait_send()
    rdB[HOPS - 1].wait_send()

    stageA[...] = partA[1].astype(jnp.float32)
    stA = pltpu.make_async_copy(
        stageA, ochunk(jnp.mod(p + 1, N_DEV), 0), storeA_sem)
    stA.start()
    stageB[...] = partB[N_DEV - 1].astype(jnp.float32)
    stB = pltpu.make_async_copy(
        stageB, ochunk(jnp.mod(p - 1, N_DEV), HC), storeB_sem)
    stB.start()

    for g in range(HOPS):
        agA[g].wait_recv()
        if g < HOPS - 1:
            agA[g + 1].start()
        stA.wait()
        stageA[...] = agA_recv[g].astype(jnp.float32)
        stA = pltpu.make_async_copy(
            stageA, ochunk(jnp.mod(p - g, N_DEV), 0), storeA_sem)
        stA.start()
        agB[g].wait_recv()
        if g < HOPS - 1:
            agB[g + 1].start()
        stB.wait()
        stageB[...] = agB_recv[g].astype(jnp.float32)
        stB = pltpu.make_async_copy(
            stageB, ochunk(jnp.mod(p + g, N_DEV), HC), storeB_sem)
        stB.start()

    for g in range(HOPS):
        agA[g].wait_send()
        agB[g].wait_send()
    stA.wait()
    stB.wait()


def kernel(x, k, Wp):
    return pl.pallas_call(
        _body,
        in_specs=[
            pl.BlockSpec(memory_space=pl.ANY),
            pl.BlockSpec(memory_space=pltpu.VMEM),
            pl.BlockSpec(memory_space=pltpu.VMEM),
        ],
        out_specs=pl.BlockSpec(memory_space=pl.ANY),
        out_shape=jax.ShapeDtypeStruct((B, S, OC), jnp.float32),
        scratch_shapes=[
            pltpu.VMEM((N_DEV, B, CHUNK, HC), jnp.bfloat16),
            pltpu.VMEM((N_DEV, B, CHUNK, HC), jnp.bfloat16),
            pltpu.VMEM((HOPS, B, CHUNK, HC), jnp.bfloat16),
            pltpu.VMEM((HOPS, B, CHUNK, HC), jnp.bfloat16),
            pltpu.VMEM((HOPS, B, CHUNK, HC), jnp.bfloat16),
            pltpu.VMEM((HOPS, B, CHUNK, HC), jnp.bfloat16),
            pltpu.VMEM((B, CHUNK, HC), jnp.float32),
            pltpu.VMEM((B, CHUNK, HC), jnp.float32),
            pltpu.VMEM((2, CHUNK + HALO, C), jnp.float32),
            pltpu.SemaphoreType.DMA((2,)),
            pltpu.SemaphoreType.DMA,
            pltpu.SemaphoreType.DMA,
            pltpu.SemaphoreType.DMA((HOPS,)),
            pltpu.SemaphoreType.DMA((HOPS,)),
            pltpu.SemaphoreType.DMA((HOPS,)),
            pltpu.SemaphoreType.DMA((HOPS,)),
            pltpu.SemaphoreType.DMA((HOPS,)),
            pltpu.SemaphoreType.DMA((HOPS,)),
            pltpu.SemaphoreType.DMA((HOPS,)),
            pltpu.SemaphoreType.DMA((HOPS,)),
        ],
        compiler_params=pltpu.CompilerParams(
            collective_id=0,
            vmem_limit_bytes=60 * 1024 * 1024,
        ),
    )(x, k, Wp)


# device time: 179045 ns/iter; 1.2621x vs baseline; 1.0517x over previous
import jax
import jax.numpy as jnp
from jax import lax
from jax.experimental import pallas as pl
from jax.experimental.pallas import tpu as pltpu

N_DEV = 4
B, S, C = 4, 2048, 1024
OC = 1024
HC = OC // 2
KT = 4
CHUNK = S // N_DEV
HALO = 8
OFF = HALO - KT + 1
HOPS = N_DEV - 1
NSUB = 2
SUBB = B // NSUB


def _body(x_ref, k_ref, Wp_ref, out_ref,
          partA, partB, rsA_recv, rsB_recv, agA_recv, agB_recv,
          stageA, stageB, xbuf,
          x_sem, storeA_sem, storeB_sem,
          rsA_send_s, rsA_recv_s, rsB_send_s, rsB_recv_s,
          agA_send_s, agA_recv_s, agB_send_s, agB_recv_s):
    p = lax.axis_index("i")
    right = jnp.mod(p + 1, N_DEV)
    left = jnp.mod(p + N_DEV - 1, N_DEV)

    barrier = pltpu.get_barrier_semaphore()
    for nbr in (left, right):
        pl.semaphore_signal(barrier, inc=1, device_id=(nbr,),
                            device_id_type=pl.DeviceIdType.MESH)
    pl.semaphore_wait(barrier, 2)

    def start_load(cglob, b, slot):
        halo_start = pl.multiple_of(
            jnp.maximum(cglob * CHUNK - HALO, 0), HALO)
        halo = pltpu.make_async_copy(
            x_ref.at[b, pl.ds(halo_start, HALO), :],
            xbuf.at[slot, pl.ds(0, HALO), :], x_sem.at[slot])
        main = pltpu.make_async_copy(
            x_ref.at[b, pl.ds(cglob * CHUNK, CHUNK), :],
            xbuf.at[slot, pl.ds(HALO, CHUNK), :], x_sem.at[slot])
        halo.start()
        main.start()
        return (halo, main, slot)

    def wait_load(ld, cglob):
        halo, main, slot = ld
        halo.wait()
        main.wait()

        @pl.when(cglob == 0)
        def _():
            xbuf[slot, :HALO, :] = jnp.zeros((HALO, C), jnp.float32)

    def compute_chunk(j, hook_mid=None):
        cglob = jnp.mod(p + j, N_DEV)
        cur = start_load(cglob, 0, 0)
        for b in range(B):
            nxt = start_load(cglob, b + 1, (b + 1) % 2) if b < B - 1 else None
            wait_load(cur, cglob)
            xv = xbuf[b % 2]
            acc = xv[OFF + KT - 1: OFF + KT - 1 + CHUNK, :] * k_ref[KT - 1, :]
            for t in range(KT - 1):
                acc = acc + xv[OFF + t: OFF + t + CHUNK, :] * k_ref[t, :]
            a = acc * (1.0 / (1.0 + jnp.exp(-acc)))
            r = jnp.dot(a, Wp_ref[...], preferred_element_type=jnp.float32)
            partA[j, b, :, :] = r[:, :HC].astype(jnp.bfloat16)
            partB[j, b, :, :] = r[:, HC:].astype(jnp.bfloat16)
            if b == SUBB - 1 and hook_mid is not None:
                hook_mid()
            cur = nxt

    def rdma(src, dst, send_sem, recv_sem, dev):
        return pltpu.make_async_remote_copy(
            src_ref=src, dst_ref=dst, send_sem=send_sem, recv_sem=recv_sem,
            device_id=(dev,), device_id_type=pl.DeviceIdType.MESH,
        )

    def sub(ref_slot, s):
        return ref_slot.at[pl.ds(s * SUBB, SUBB)]

    def osub(c, off, s):
        return out_ref.at[pl.ds(s * SUBB, SUBB),
                          pl.ds(c * CHUNK, CHUNK), pl.ds(off, HC)]

    rdA = [[rdma(sub(partA.at[(N_DEV - h) % N_DEV], s),
                 sub(rsA_recv.at[h], s),
                 rsA_send_s.at[h, s], rsA_recv_s.at[h, s], right)
            for s in range(NSUB)] for h in range(HOPS)]
    rdB = [[rdma(sub(partB.at[h], s), sub(rsB_recv.at[h], s),
                 rsB_send_s.at[h, s], rsB_recv_s.at[h, s], left)
            for s in range(NSUB)] for h in range(HOPS)]

    def _hop0_sub0():
        rdA[0][0].start()
        rdB[0][0].start()

    compute_chunk(0, hook_mid=_hop0_sub0)
    rdA[0][1].start()
    rdB[0][1].start()
    compute_chunk(N_DEV - 1)
    compute_chunk(1)

    for h in range(HOPS):
        sA = N_DEV - 1 - h
        sB = h + 1
        for s in range(NSUB):
            rows = slice(s * SUBB, (s + 1) * SUBB)
            rdA[h][s].wait_recv()
            partA[sA, rows] = partA[sA, rows] + rsA_recv[h, rows]
            if h < HOPS - 1:
                rdA[h + 1][s].start()
            rdB[h][s].wait_recv()
            partB[sB, rows] = partB[sB, rows] + rsB_recv[h, rows]
            if h < HOPS - 1:
                rdB[h + 1][s].start()
        if h < HOPS - 1:
            for s in range(NSUB):
                rdA[h][s].wait_send()
                rdB[h][s].wait_send()
        if h == 0:
            compute_chunk(2)

    agA = [[rdma(sub(partA.at[1], s) if g == 0
                 else sub(agA_recv.at[g - 1], s),
                 sub(agA_recv.at[g], s),
                 agA_send_s.at[g, s], agA_recv_s.at[g, s], right)
            for s in range(NSUB)] for g in range(HOPS)]
    agB = [[rdma(sub(partB.at[N_DEV - 1], s) if g == 0
                 else sub(agB_recv.at[g - 1], s),
                 sub(agB_recv.at[g], s),
                 agB_send_s.at[g, s], agB_recv_s.at[g, s], left)
            for s in range(NSUB)] for g in range(HOPS)]

    for s in range(NSUB):
        agA[0][s].start()
        agB[0][s].start()
    for s in range(NSUB):
        rdA[HOPS - 1][s].wait_send()
        rdB[HOPS - 1][s].wait_send()

    stA = [None] * NSUB
    stB = [None] * NSUB
    for s in range(NSUB):
        rows = slice(s * SUBB, (s + 1) * SUBB)
        stageA[rows] = partA[1, rows].astype(jnp.float32)
        stA[s] = pltpu.make_async_copy(
            sub(stageA, s), osub(jnp.mod(p + 1, N_DEV), 0, s),
            storeA_sem.at[s])
        stA[s].start()
        stageB[rows] = partB[N_DEV - 1, rows].astype(jnp.float32)
        stB[s] = pltpu.make_async_copy(
            sub(stageB, s), osub(jnp.mod(p - 1, N_DEV), HC, s),
            storeB_sem.at[s])
        stB[s].start()

    for g in range(HOPS):
        for s in range(NSUB):
            rows = slice(s * SUBB, (s + 1) * SUBB)
            agA[g][s].wait_recv()
            if g < HOPS - 1:
                agA[g + 1][s].start()
            stA[s].wait()
            stageA[rows] = agA_recv[g, rows].astype(jnp.float32)
            stA[s] = pltpu.make_async_copy(
                sub(stageA, s), osub(jnp.mod(p - g, N_DEV), 0, s),
                storeA_sem.at[s])
            stA[s].start()
            agB[g][s].wait_recv()
            if g < HOPS - 1:
                agB[g + 1][s].start()
            stB[s].wait()
            stageB[rows] = agB_recv[g, rows].astype(jnp.float32)
            stB[s] = pltpu.make_async_copy(
                sub(stageB, s), osub(jnp.mod(p + g, N_DEV), HC, s),
                storeB_sem.at[s])
            stB[s].start()

    for g in range(HOPS):
        for s in range(NSUB):
            agA[g][s].wait_send()
            agB[g][s].wait_send()
    for s in range(NSUB):
        stA[s].wait()
        stB[s].wait()


def kernel(x, k, Wp):
    return pl.pallas_call(
        _body,
        in_specs=[
            pl.BlockSpec(memory_space=pl.ANY),
            pl.BlockSpec(memory_space=pltpu.VMEM),
            pl.BlockSpec(memory_space=pltpu.VMEM),
        ],
        out_specs=pl.BlockSpec(memory_space=pl.ANY),
        out_shape=jax.ShapeDtypeStruct((B, S, OC), jnp.float32),
        scratch_shapes=[
            pltpu.VMEM((N_DEV, B, CHUNK, HC), jnp.bfloat16),
            pltpu.VMEM((N_DEV, B, CHUNK, HC), jnp.bfloat16),
            pltpu.VMEM((HOPS, B, CHUNK, HC), jnp.bfloat16),
            pltpu.VMEM((HOPS, B, CHUNK, HC), jnp.bfloat16),
            pltpu.VMEM((HOPS, B, CHUNK, HC), jnp.bfloat16),
            pltpu.VMEM((HOPS, B, CHUNK, HC), jnp.bfloat16),
            pltpu.VMEM((B, CHUNK, HC), jnp.float32),
            pltpu.VMEM((B, CHUNK, HC), jnp.float32),
            pltpu.VMEM((2, CHUNK + HALO, C), jnp.float32),
            pltpu.SemaphoreType.DMA((2,)),
            pltpu.SemaphoreType.DMA((NSUB,)),
            pltpu.SemaphoreType.DMA((NSUB,)),
            pltpu.SemaphoreType.DMA((HOPS, NSUB)),
            pltpu.SemaphoreType.DMA((HOPS, NSUB)),
            pltpu.SemaphoreType.DMA((HOPS, NSUB)),
            pltpu.SemaphoreType.DMA((HOPS, NSUB)),
            pltpu.SemaphoreType.DMA((HOPS, NSUB)),
            pltpu.SemaphoreType.DMA((HOPS, NSUB)),
            pltpu.SemaphoreType.DMA((HOPS, NSUB)),
            pltpu.SemaphoreType.DMA((HOPS, NSUB)),
        ],
        compiler_params=pltpu.CompilerParams(
            collective_id=0,
            vmem_limit_bytes=60 * 1024 * 1024,
        ),
    )(x, k, Wp)
